# baseline (speedup 1.0000x reference)
import functools

import jax
import jax.numpy as jnp
from jax import lax
from jax.experimental import pallas as pl
from jax.experimental.pallas import tpu as pltpu
from jax.experimental.pallas import tpu_sc as plsc

D = 384
N_NODE = 128
N_DEPTH = 32
N_TOK = 4 * 8192

NC = 2
NS = 16
L = 16
NW = NC * NS
TOK_W = N_TOK // NW
CH = 64
NCH = TOK_W // CH
NB = 2
NG = CH // L
UNROLL = 4


def _sc_body(nid_hbm, did_hbm, ntab_hbm, dtab_hbm, out_hbm,
             ntab_v, dtab_v, nidx_v, didx_v, rows_v, ssem):
    wid = lax.axis_index("s") * NC + lax.axis_index("c")
    base = wid * TOK_W

    pltpu.sync_copy(ntab_hbm, ntab_v)
    pltpu.sync_copy(dtab_hbm, dtab_v)
    pltpu.sync_copy(nid_hbm.at[pl.ds(base, TOK_W)], nidx_v)
    pltpu.sync_copy(did_hbm.at[pl.ds(base, TOK_W)], didx_v)

    lanes = jax.lax.iota(jnp.int32, L)

    def _compute(c):
        buf = rows_v.at[c % NB]
        nrows = []
        drows = []
        srows = []
        for g in range(NG):
            t0 = c * CH + g * L
            nrows.append(nidx_v[pl.ds(t0, L)])
            drows.append(didx_v[pl.ds(t0, L)])
            srows.append(lanes + g * L)

        def _col(i, carry):
            col0 = i * UNROLL
            for u in range(UNROLL):
                colv = jnp.zeros((L,), jnp.int32) + (col0 + u)
                for g in range(NG):
                    a = plsc.load_gather(ntab_v, [nrows[g], colv])
                    b = plsc.load_gather(dtab_v, [drows[g], colv])
                    plsc.store_scatter(buf, [srows[g], colv], a + b)
            return carry

        lax.fori_loop(0, D // UNROLL, _col, 0)

    def _scatter(c):
        return pltpu.async_copy(
            rows_v.at[c % NB], out_hbm.at[pl.ds(base + c * CH, CH)], ssem)

    scatters = [None] * NCH
    for c in range(NCH):
        if c - NB >= 0:
            scatters[c - NB].wait()
        _compute(c)
        scatters[c] = _scatter(c)
    for c in range(NCH - NB, NCH):
        scatters[c].wait()


@jax.jit
def _run(node_ids, depth_ids, ntab_flat, dtab_flat):
    k = functools.partial(
        pl.kernel,
        out_type=jax.ShapeDtypeStruct((N_TOK, D), jnp.float32),
        mesh=plsc.VectorSubcoreMesh(core_axis_name="c", subcore_axis_name="s"),
        compiler_params=pltpu.CompilerParams(needs_layout_passes=False),
        scratch_types=[
            pltpu.VMEM((N_NODE, D), jnp.float32),
            pltpu.VMEM((N_DEPTH, D), jnp.float32),
            pltpu.VMEM((TOK_W,), jnp.int32),
            pltpu.VMEM((TOK_W,), jnp.int32),
            pltpu.VMEM((NB, CH, D), jnp.float32),
            pltpu.SemaphoreType.DMA,
        ],
    )(_sc_body)
    return k(node_ids, depth_ids, ntab_flat, dtab_flat)


def kernel(node_type_ids, depth_ids, node_table, depth_table):
    b, t = node_type_ids.shape
    nid = node_type_ids.reshape(-1).astype(jnp.int32)
    did = depth_ids.reshape(-1).astype(jnp.int32)
    out = _run(nid, did, node_table, depth_table)
    return out.reshape(b, t, D)


# speedup vs baseline: 12.4466x; 12.4466x over previous
import functools

import jax
import jax.numpy as jnp
from jax import lax
from jax.experimental import pallas as pl
from jax.experimental.pallas import tpu as pltpu
from jax.experimental.pallas import tpu_sc as plsc

D = 384
N_NODE = 128
N_DEPTH = 32
N_TOK = 4 * 8192

NC = 2
NS = 16
L = 16
NW = NC * NS
TOK_W = N_TOK // NW
CH = 64
NCH = TOK_W // CH
NB = 5


def _prep_body(node_ref, depth_ref, nid_ref, did_ref, ctab_ref, cidx_ref):
    node = node_ref[...]
    depth = depth_ref[...]
    ctab_ref[...] = node[:, None, :] + depth[None, :, :]
    cidx_ref[...] = nid_ref[...] * N_DEPTH + did_ref[...]


def _prep(node_table, depth_table, nid, did):
    ctab, cidx = pl.pallas_call(
        _prep_body,
        out_shape=(
            jax.ShapeDtypeStruct((N_NODE, N_DEPTH, D), jnp.float32),
            jax.ShapeDtypeStruct(nid.shape, jnp.int32),
        ),
    )(node_table, depth_table, nid, did)
    return ctab.reshape(N_NODE * N_DEPTH, D), cidx.reshape(-1)


def _sc_body(cidx_hbm, ctab_hbm, out_hbm, cidx_v, rows_v, gsem, ssem):
    wid = lax.axis_index("s") * NC + lax.axis_index("c")
    base = wid * TOK_W
    pltpu.sync_copy(cidx_hbm.at[pl.ds(base, TOK_W)], cidx_v)

    def _gather(c):
        idx = cidx_v.at[pl.ds(c * CH, CH)]
        return pltpu.async_copy(ctab_hbm.at[idx], rows_v.at[c % NB], gsem)

    def _scatter(c):
        return pltpu.async_copy(
            rows_v.at[c % NB], out_hbm.at[pl.ds(base + c * CH, CH)], ssem)

    gathers = [None] * NCH
    scatters = [None] * NCH
    for c in range(NB - 1):
        gathers[c] = _gather(c)
    for c in range(NCH):
        gathers[c].wait()
        nxt = c + NB - 1
        if nxt < NCH:
            if c - 1 >= 0:
                scatters[c - 1].wait()
            gathers[nxt] = _gather(nxt)
        scatters[c] = _scatter(c)
    for c in range(NCH - NB, NCH):
        if c >= 0:
            scatters[c].wait()


@jax.jit
def _run(node_ids, depth_ids, node_table, depth_table):
    ctab, cidx = _prep(node_table, depth_table, node_ids, depth_ids)
    k = functools.partial(
        pl.kernel,
        out_type=jax.ShapeDtypeStruct((N_TOK, D), jnp.float32),
        mesh=plsc.VectorSubcoreMesh(core_axis_name="c", subcore_axis_name="s"),
        scratch_types=[
            pltpu.VMEM((TOK_W,), jnp.int32),
            pltpu.VMEM((NB, CH, D), jnp.float32),
            pltpu.SemaphoreType.DMA,
            pltpu.SemaphoreType.DMA,
        ],
    )(_sc_body)
    return k(cidx, ctab)


def kernel(node_type_ids, depth_ids, node_table, depth_table):
    b, t = node_type_ids.shape
    nid = node_type_ids.astype(jnp.int32)
    did = depth_ids.astype(jnp.int32)
    out = _run(nid, did, node_table, depth_table)
    return out.reshape(b, t, D)


# speedup vs baseline: 12.4822x; 1.0029x over previous
import functools

import jax
import jax.numpy as jnp
from jax import lax
from jax.experimental import pallas as pl
from jax.experimental.pallas import tpu as pltpu
from jax.experimental.pallas import tpu_sc as plsc

D = 384
N_NODE = 128
N_DEPTH = 32
N_TOK = 4 * 8192

NC = 2
NS = 16
L = 16
NW = NC * NS
TOK_W = N_TOK // NW
CH = 64
NCH = TOK_W // CH
NB = 5


def _prep_body(node_ref, depth_ref, nid_ref, did_ref, ctab_ref, cidx_ref):
    node = node_ref[...]
    depth = depth_ref[...]
    ctab_ref[...] = node[:, None, :] + depth[None, :, :]
    cidx_ref[...] = nid_ref[...] * N_DEPTH + did_ref[...]


def _prep(node_table, depth_table, nid, did):
    ctab, cidx = pl.pallas_call(
        _prep_body,
        out_shape=(
            jax.ShapeDtypeStruct((N_NODE, N_DEPTH, D), jnp.float32),
            jax.ShapeDtypeStruct(nid.shape, jnp.int32),
        ),
    )(node_table, depth_table, nid, did)
    return ctab.reshape(N_NODE * N_DEPTH, D), cidx.reshape(-1)


def _sc_body(cidx_hbm, ctab_hbm, out_hbm, cidx_v, rows_v, gsem, ssem):
    wid = lax.axis_index("s") * NC + lax.axis_index("c")
    base = wid * TOK_W
    pltpu.sync_copy(cidx_hbm.at[pl.ds(base, TOK_W)], cidx_v)

    def _gather(c):
        idx = cidx_v.at[pl.ds(c * CH, CH)]
        return pltpu.async_copy(ctab_hbm.at[idx], rows_v.at[c % NB], gsem)

    def _scatter(c):
        return pltpu.async_copy(
            rows_v.at[c % NB], out_hbm.at[pl.ds(base + c * CH, CH)], ssem)

    gathers = [None] * NCH
    scatters = [None] * NCH
    for c in range(NB - 1):
        gathers[c] = _gather(c)
    for c in range(NCH):
        gathers[c].wait()
        nxt = c + NB - 1
        if nxt < NCH:
            if c - 1 >= 0:
                scatters[c - 1].wait()
            gathers[nxt] = _gather(nxt)
        scatters[c] = _scatter(c)
    for c in range(NCH - NB, NCH):
        if c >= 0:
            scatters[c].wait()


@jax.jit
def _run(node_ids, depth_ids, node_table, depth_table):
    ctab, cidx = _prep(node_table, depth_table, node_ids, depth_ids)
    k = functools.partial(
        pl.kernel,
        out_type=jax.ShapeDtypeStruct((N_TOK, D), jnp.float32),
        mesh=plsc.VectorSubcoreMesh(core_axis_name="c", subcore_axis_name="s"),
        compiler_params=pltpu.CompilerParams(
            disable_bounds_checks=True, disable_semaphore_checks=True),
        scratch_types=[
            pltpu.VMEM((TOK_W,), jnp.int32),
            pltpu.VMEM((NB, CH, D), jnp.float32),
            pltpu.SemaphoreType.DMA,
            pltpu.SemaphoreType.DMA,
        ],
    )(_sc_body)
    return k(cidx, ctab)


def kernel(node_type_ids, depth_ids, node_table, depth_table):
    b, t = node_type_ids.shape
    nid = node_type_ids.astype(jnp.int32)
    did = depth_ids.astype(jnp.int32)
    out = _run(nid, did, node_table, depth_table)
    return out.reshape(b, t, D)
